# baseline (speedup 1.0000x reference)
import functools

import jax
import jax.numpy as jnp
from jax import lax
from jax.experimental import pallas as pl
from jax.experimental.pallas import tpu as pltpu
from jax.experimental.pallas import tpu_sc as plsc

B, C, H, W = 32, 64, 32, 32
HW = H * W
N = B * HW
V = 8192
RB = 256
QB = HW // RB
NB = N // RB
BETA = 0.25
RESI = 0.5

NWORK = 32
IDX_ROWS = N // 128
ROWS_PER_W = IDX_ROWS // NWORK
VSLICE = V // NWORK


CK = 2048


def _dist_body(f_ref, emb_ref, en_ref, idx_ref):
    fbt = f_ref[0].T
    xn = jnp.sum(fbt * fbt, axis=1, keepdims=True)
    run_mn = None
    run_id = None
    vid0 = lax.broadcasted_iota(jnp.int32, (RB, CK), 1)
    for c in range(V // CK):
        e = emb_ref[c * CK:(c + 1) * CK, :]
        m = lax.dot_general(fbt, e, (((1,), (1,)), ((), ())),
                            preferred_element_type=jnp.float32)
        d = (xn + en_ref[:, c * CK:(c + 1) * CK]) - 2.0 * m
        mn_c = jnp.min(d, axis=1, keepdims=True)
        id_c = jnp.min(jnp.where(d == mn_c, vid0 + (c * CK), V),
                       axis=1).astype(jnp.int32)
        if c == 0:
            run_mn, run_id = mn_c, id_c
        else:
            better = mn_c < run_mn
            run_id = jnp.where(better[:, 0], id_c, run_id)
            run_mn = jnp.where(better, mn_c, run_mn)
    idx_ref[...] = run_id.reshape(1, 1, RB)


_dist = pl.pallas_call(
    _dist_body,
    grid=(B, QB),
    in_specs=[
        pl.BlockSpec((1, C, RB), lambda b, q: (b, 0, q)),
        pl.BlockSpec((V, C), lambda b, q: (0, 0)),
        pl.BlockSpec((1, V), lambda b, q: (0, 0)),
    ],
    out_specs=pl.BlockSpec((1, 1, RB), lambda b, q: (b * QB + q, 0, 0)),
    out_shape=jax.ShapeDtypeStruct((NB, 1, RB), jnp.int32),
)


def _sc_body(idx_hbm, emb_hbm, h_out, hit_out, idx_all, idxb, rows, hist, sem):
    wid = lax.axis_index("s") * 2 + lax.axis_index("c")
    base = wid * ROWS_PER_W
    half = ROWS_PER_W // 2

    pltpu.sync_copy(idx_hbm, idx_all)
    pltpu.sync_copy(idx_hbm.at[pl.ds(base, ROWS_PER_W)], idxb)

    lo = wid * VSLICE
    zeros16 = jnp.zeros((16,), jnp.float32)
    ones16 = jnp.ones((16,), jnp.float32)
    for k in range(VSLICE // 16):
        hist[pl.ds(k * 16, 16)] = zeros16

    def scan_rows(r0, rn):
        def row_body(r, carry):
            for k in range(128 // 16):
                v = idx_all[r, pl.ds(k * 16, 16)]
                m = (v >= lo) & (v < lo + VSLICE)
                plsc.addupdate_scatter(hist, [v - lo], ones16, mask=m)
            return carry
        lax.fori_loop(r0, r0 + rn, row_body, 0)

    for r in range(2):
        cps = [pltpu.async_copy(emb_hbm.at[idxb.at[r * half + j]],
                                rows.at[j], sem)
               for j in range(half)]
        scan_rows(r * (IDX_ROWS // 2), IDX_ROWS // 2)
        for cp in cps:
            cp.wait()
        pltpu.sync_copy(rows, h_out.at[pl.ds(base + r * half, half)])

    pltpu.sync_copy(hist, hit_out.at[pl.ds(lo, VSLICE)])


@functools.cache
def _sc_gather():
    return pl.kernel(
        _sc_body,
        mesh=plsc.VectorSubcoreMesh(core_axis_name="c", subcore_axis_name="s"),
        out_type=[
            jax.ShapeDtypeStruct((IDX_ROWS, 128, 128), jnp.float32),
            jax.ShapeDtypeStruct((V,), jnp.float32),
        ],
        scratch_types=[
            pltpu.VMEM((IDX_ROWS, 128), jnp.int32),
            pltpu.VMEM((ROWS_PER_W, 128), jnp.int32),
            pltpu.VMEM((ROWS_PER_W // 2, 128, 128), jnp.float32),
            pltpu.VMEM((VSLICE,), jnp.float32),
            pltpu.SemaphoreType.DMA,
        ],
        compiler_params=pltpu.CompilerParams(needs_layout_passes=False),
    )


def _conv_body(h_ref, f_ref, wt_ref, b_ref, out_ref, loss_ref):
    bi = pl.program_id(0)
    x = h_ref[0, :, :C]
    wcol = lax.broadcasted_iota(jnp.int32, (HW, C), 0) % W
    acc = jnp.zeros((HW, C), jnp.float32)
    for ky in range(3):
        dy = ky - 1
        for kx in range(3):
            dx = kx - 1
            s = dy * W + dx
            if s > 0:
                patch = jnp.concatenate(
                    [x[s:], jnp.zeros((s, C), jnp.float32)], axis=0)
            elif s < 0:
                patch = jnp.concatenate(
                    [jnp.zeros((-s, C), jnp.float32), x[:HW + s]], axis=0)
            else:
                patch = x
            if dx == 1:
                patch = jnp.where(wcol == W - 1, 0.0, patch)
            elif dx == -1:
                patch = jnp.where(wcol == 0, 0.0, patch)
            acc = acc + jnp.dot(patch, wt_ref[ky, kx],
                                preferred_element_type=jnp.float32)
    fh = x * (1.0 - RESI) + (acc + b_ref[...]) * RESI
    fh_t = fh.T
    out_ref[0] = fh_t
    dlt = fh_t - f_ref[0]
    part = jnp.sum(dlt * dlt).reshape(1, 1)

    @pl.when(bi == 0)
    def _():
        loss_ref[...] = jnp.zeros((1, 1), jnp.float32)

    loss_ref[...] += part

    @pl.when(bi == B - 1)
    def _():
        loss_ref[...] = loss_ref[...] * ((1.0 + BETA) / (B * C * HW))


_conv = pl.pallas_call(
    _conv_body,
    grid=(B,),
    in_specs=[
        pl.BlockSpec((1, HW, 128), lambda b: (b, 0, 0)),
        pl.BlockSpec((1, C, HW), lambda b: (b, 0, 0)),
        pl.BlockSpec((3, 3, C, C), lambda b: (0, 0, 0, 0)),
        pl.BlockSpec((1, C), lambda b: (0, 0)),
    ],
    out_specs=[
        pl.BlockSpec((1, C, HW), lambda b: (b, 0, 0)),
        pl.BlockSpec((1, 1), lambda b: (0, 0)),
    ],
    out_shape=[
        jax.ShapeDtypeStruct((B, C, HW), jnp.float32),
        jax.ShapeDtypeStruct((1, 1), jnp.float32),
    ],
)


def kernel(f_BChw, emb, conv_w, conv_b):
    f3 = f_BChw.reshape(B, C, HW)
    en_row = jnp.sum(emb * emb, axis=1)[None]
    idx_blocks = _dist(f3, emb, en_row)
    idx2 = idx_blocks.reshape(IDX_ROWS, 128)
    emb_pad = jnp.pad(emb, ((0, 0), (0, 128 - C)))
    h3, hit_V = _sc_gather()(idx2, emb_pad)
    h_b = h3.reshape(B, HW, 128)
    wt = jnp.transpose(conv_w, (2, 3, 1, 0))
    f_hat3, loss11 = _conv(h_b, f3, wt, conv_b.reshape(1, C))
    return (f_hat3.reshape(B, C, H, W), loss11[0, 0], hit_V)


# speedup vs baseline: 1.1446x; 1.1446x over previous
import functools

import jax
import jax.numpy as jnp
from jax import lax
from jax.experimental import pallas as pl
from jax.experimental.pallas import tpu as pltpu
from jax.experimental.pallas import tpu_sc as plsc

B, C, H, W = 32, 64, 32, 32
HW = H * W
N = B * HW
V = 8192
RB = 256
QB = HW // RB
BH = B // 2
NH = BH * HW
NBH = NH // RB
BETA = 0.25
RESI = 0.5

NWORK = 32
IDX_ROWS = NH // 128
ROWS_PER_W = IDX_ROWS // NWORK
VSLICE = V // NWORK


def _dist_body(f_ref, emb_ref, en_ref, idx_ref):
    fbt = f_ref[0].T
    m = lax.dot_general(fbt, emb_ref[...], (((1,), (1,)), ((), ())),
                        preferred_element_type=jnp.float32)
    xn = jnp.sum(fbt * fbt, axis=1, keepdims=True)
    d = (xn + en_ref[...]) - 2.0 * m
    mn = jnp.min(d, axis=1, keepdims=True)
    vids = lax.broadcasted_iota(jnp.int32, (RB, V), 1)
    idx = jnp.min(jnp.where(d == mn, vids, V), axis=1).astype(jnp.int32)
    idx_ref[...] = idx.reshape(1, 1, RB)


def _make_dist(b_ofs):
    return pl.pallas_call(
        _dist_body,
        grid=(BH, QB),
        in_specs=[
            pl.BlockSpec((1, C, RB), lambda b, q: (b + b_ofs, 0, q)),
            pl.BlockSpec((V, C), lambda b, q: (0, 0)),
            pl.BlockSpec((1, V), lambda b, q: (0, 0)),
        ],
        out_specs=pl.BlockSpec((1, 1, RB), lambda b, q: (b * QB + q, 0, 0)),
        out_shape=jax.ShapeDtypeStruct((NBH, 1, RB), jnp.int32),
    )


_dist0 = _make_dist(0)
_dist1 = _make_dist(BH)


def _sc_body(idx_hbm, emb_hbm, h_out, hit_out, idx_all, idxb, rows, hist, sem):
    wid = lax.axis_index("s") * 2 + lax.axis_index("c")
    base = wid * ROWS_PER_W

    pltpu.sync_copy(idx_hbm, idx_all)
    pltpu.sync_copy(idx_hbm.at[pl.ds(base, ROWS_PER_W)], idxb)

    lo = wid * VSLICE
    zeros16 = jnp.zeros((16,), jnp.float32)
    ones16 = jnp.ones((16,), jnp.float32)
    for k in range(VSLICE // 16):
        hist[pl.ds(k * 16, 16)] = zeros16

    cps = [pltpu.async_copy(emb_hbm.at[idxb.at[j]], rows.at[j], sem)
           for j in range(ROWS_PER_W)]

    def row_body(r, carry):
        for k in range(128 // 16):
            v = idx_all[r, pl.ds(k * 16, 16)]
            m = (v >= lo) & (v < lo + VSLICE)
            plsc.addupdate_scatter(hist, [v - lo], ones16, mask=m)
        return carry

    lax.fori_loop(0, IDX_ROWS, row_body, 0)

    for cp in cps:
        cp.wait()
    pltpu.sync_copy(rows, h_out.at[pl.ds(base, ROWS_PER_W)])
    pltpu.sync_copy(hist, hit_out.at[pl.ds(lo, VSLICE)])


@functools.cache
def _sc_gather():
    return pl.kernel(
        _sc_body,
        mesh=plsc.VectorSubcoreMesh(core_axis_name="c", subcore_axis_name="s"),
        out_type=[
            jax.ShapeDtypeStruct((IDX_ROWS, 128, 128), jnp.float32),
            jax.ShapeDtypeStruct((V,), jnp.float32),
        ],
        scratch_types=[
            pltpu.VMEM((IDX_ROWS, 128), jnp.int32),
            pltpu.VMEM((ROWS_PER_W, 128), jnp.int32),
            pltpu.VMEM((ROWS_PER_W, 128, 128), jnp.float32),
            pltpu.VMEM((VSLICE,), jnp.float32),
            pltpu.SemaphoreType.DMA,
        ],
        compiler_params=pltpu.CompilerParams(needs_layout_passes=False),
    )


def _conv_common(h_ref, f_ref, wt_ref, b_ref):
    x = h_ref[0, :, :C]
    wcol = lax.broadcasted_iota(jnp.int32, (HW, C), 0) % W
    acc = jnp.zeros((HW, C), jnp.float32)
    for ky in range(3):
        dy = ky - 1
        for kx in range(3):
            dx = kx - 1
            s = dy * W + dx
            if s > 0:
                patch = jnp.concatenate(
                    [x[s:], jnp.zeros((s, C), jnp.float32)], axis=0)
            elif s < 0:
                patch = jnp.concatenate(
                    [jnp.zeros((-s, C), jnp.float32), x[:HW + s]], axis=0)
            else:
                patch = x
            if dx == 1:
                patch = jnp.where(wcol == W - 1, 0.0, patch)
            elif dx == -1:
                patch = jnp.where(wcol == 0, 0.0, patch)
            acc = acc + jnp.dot(patch, wt_ref[ky, kx],
                                preferred_element_type=jnp.float32)
    fh = x * (1.0 - RESI) + (acc + b_ref[...]) * RESI
    fh_t = fh.T
    dlt = fh_t - f_ref[0]
    return fh_t, jnp.sum(dlt * dlt).reshape(1, 1)


def _conv0_body(h_ref, f_ref, wt_ref, b_ref, out_ref, loss_ref):
    bi = pl.program_id(0)
    fh_t, part = _conv_common(h_ref, f_ref, wt_ref, b_ref)
    out_ref[0] = fh_t

    @pl.when(bi == 0)
    def _():
        loss_ref[...] = jnp.zeros((1, 1), jnp.float32)

    loss_ref[...] += part


def _conv1_body(h_ref, f_ref, wt_ref, b_ref, hp0_ref, hp1_ref, l0_ref,
                out_ref, loss_ref, hit_ref):
    bi = pl.program_id(0)
    fh_t, part = _conv_common(h_ref, f_ref, wt_ref, b_ref)
    out_ref[0] = fh_t

    @pl.when(bi == 0)
    def _():
        loss_ref[...] = l0_ref[...]
        hit_ref[...] = hp0_ref[...] + hp1_ref[...]

    loss_ref[...] += part

    @pl.when(bi == BH - 1)
    def _():
        loss_ref[...] = loss_ref[...] * ((1.0 + BETA) / (B * C * HW))


_conv0 = pl.pallas_call(
    _conv0_body,
    grid=(BH,),
    in_specs=[
        pl.BlockSpec((1, HW, 128), lambda b: (b, 0, 0)),
        pl.BlockSpec((1, C, HW), lambda b: (b, 0, 0)),
        pl.BlockSpec((3, 3, C, C), lambda b: (0, 0, 0, 0)),
        pl.BlockSpec((1, C), lambda b: (0, 0)),
    ],
    out_specs=[
        pl.BlockSpec((1, C, HW), lambda b: (b, 0, 0)),
        pl.BlockSpec((1, 1), lambda b: (0, 0)),
    ],
    out_shape=[
        jax.ShapeDtypeStruct((BH, C, HW), jnp.float32),
        jax.ShapeDtypeStruct((1, 1), jnp.float32),
    ],
)

_conv1 = pl.pallas_call(
    _conv1_body,
    grid=(BH,),
    in_specs=[
        pl.BlockSpec((1, HW, 128), lambda b: (b, 0, 0)),
        pl.BlockSpec((1, C, HW), lambda b: (b + BH, 0, 0)),
        pl.BlockSpec((3, 3, C, C), lambda b: (0, 0, 0, 0)),
        pl.BlockSpec((1, C), lambda b: (0, 0)),
        pl.BlockSpec((1, V), lambda b: (0, 0)),
        pl.BlockSpec((1, V), lambda b: (0, 0)),
        pl.BlockSpec((1, 1), lambda b: (0, 0)),
    ],
    out_specs=[
        pl.BlockSpec((1, C, HW), lambda b: (b, 0, 0)),
        pl.BlockSpec((1, 1), lambda b: (0, 0)),
        pl.BlockSpec((1, V), lambda b: (0, 0)),
    ],
    out_shape=[
        jax.ShapeDtypeStruct((BH, C, HW), jnp.float32),
        jax.ShapeDtypeStruct((1, 1), jnp.float32),
        jax.ShapeDtypeStruct((1, V), jnp.float32),
    ],
)


def kernel(f_BChw, emb, conv_w, conv_b):
    f3 = f_BChw.reshape(B, C, HW)
    en_row = jnp.sum(emb * emb, axis=1)[None]
    emb_pad = jnp.pad(emb, ((0, 0), (0, 128 - C)))
    wt = jnp.transpose(conv_w, (2, 3, 1, 0))
    cb1 = conv_b.reshape(1, C)
    sc = _sc_gather()

    idx0 = _dist0(f3, emb, en_row)
    h0, hit0 = sc(idx0.reshape(IDX_ROWS, 128), emb_pad)
    idx1 = _dist1(f3, emb, en_row)
    h1, hit1 = sc(idx1.reshape(IDX_ROWS, 128), emb_pad)
    fh0, l0 = _conv0(h0.reshape(BH, HW, 128), f3, wt, cb1)
    fh1, loss11, hitO = _conv1(h1.reshape(BH, HW, 128), f3, wt, cb1,
                               hit0.reshape(1, V), hit1.reshape(1, V), l0)
    f_hat = jnp.concatenate([fh0, fh1], axis=0).reshape(B, C, H, W)
    return (f_hat, loss11[0, 0], hitO.reshape(V))
